# baseline (speedup 1.0000x reference)
import jax
import jax.numpy as jnp
from jax.experimental import pallas as pl
from jax.experimental.pallas import tpu as pltpu

_BN = 1024


def _body(idx_ref, h_ref, cv_ref, o_ref):
    o_ref[...] = h_ref[...] + cv_ref[0]


def kernel(hidden_states, affective_state_index, control_vectors):
    n, d = hidden_states.shape
    k = control_vectors.shape[0]
    idx = jnp.asarray(affective_state_index, jnp.int32).reshape(1)
    cv3 = control_vectors.reshape(k, 1, d)
    return pl.pallas_call(
        _body,
        grid_spec=pltpu.PrefetchScalarGridSpec(
            num_scalar_prefetch=1,
            grid=(n // _BN,),
            in_specs=[
                pl.BlockSpec((_BN, d), lambda i, idx_ref: (i, 0)),
                pl.BlockSpec((1, 1, d), lambda i, idx_ref: (idx_ref[0], 0, 0)),
            ],
            out_specs=pl.BlockSpec((_BN, d), lambda i, idx_ref: (i, 0)),
        ),
        out_shape=jax.ShapeDtypeStruct((n, d), hidden_states.dtype),
    )(idx, hidden_states, cv3)
